# baseline (speedup 1.0000x reference)
import jax
import jax.numpy as jnp
import numpy as np
from jax.experimental import pallas as pl
from jax.experimental.pallas import tpu as pltpu

N_TIME, PER_DIM = 6, 6

_INV_PI = np.float32(0.3183098861837907)
_PI_HI = np.float32(3.140625)
_PI_LO = np.float32(9.676535897932795e-4)
_S3 = np.float32(-1.6665861e-01)
_S5 = np.float32(8.3121910e-03)
_S7 = np.float32(-1.8497128e-04)


def _fast_sin(x):
    kf = jnp.round(x * _INV_PI)
    r = x - kf * _PI_HI
    r = r - kf * _PI_LO
    r2 = r * r
    p = r + r * (r2 * (_S3 + r2 * (_S5 + r2 * _S7)))
    odd = jax.lax.shift_left(kf.astype(jnp.int32), np.int32(31))
    return jax.lax.bitcast_convert_type(
        jax.lax.bitcast_convert_type(p, jnp.int32) ^ odd, jnp.float32)


def _body(vr_ref, feat_ref, w_ref, b_ref, le_ref, out_ref, vid_ref):
    bb = pl.program_id(0)
    d_in, rows, lpr = vr_ref.shape[1], vr_ref.shape[2], vr_ref.shape[3]
    flat = feat_ref.shape[2]
    d_model = flat // lpr
    lng = rows * lpr

    xa = feat_ref[0] * w_ref[...] + b_ref[...]
    m = jax.lax.broadcasted_iota(jnp.int32, (rows, flat), 1)
    c = m % d_model
    sinsel = (c >= 1) & (c <= N_TIME * PER_DIM) & ((c - 1) % PER_DIM != 0)
    base = le_ref[...] + jnp.where(sinsel, _fast_sin(xa), xa)

    q = jax.lax.broadcasted_iota(jnp.int32, (lpr, flat), 0)
    mc = jax.lax.broadcasted_iota(jnp.int32, (lpr, flat), 1)
    spread = (mc == q * d_model).astype(jnp.float32)
    for dd in range(d_in):
        out_ref[0, dd * rows:(dd + 1) * rows, :] = base + jnp.dot(
            vr_ref[0, dd], spread, preferred_element_type=jnp.float32)

    @pl.when(bb == 0)
    def _fill_vid():
        t = jax.lax.broadcasted_iota(jnp.int32, vid_ref.shape, 1)
        vid_ref[...] = t // lng


def kernel(input, dates, cmax, time_w, time_b, local_emb, space_emb):
    b, length, d_input = input.shape
    d_model = local_emb.shape[1]
    n_time, per_dim = time_w.shape
    nt = n_time * per_dim
    lpr = 16
    flat = lpr * d_model
    rows = length // lpr

    feats = jnp.concatenate(
        [jnp.zeros((b, length, 1), jnp.float32),
         jnp.repeat(dates, per_dim, axis=-1), cmax],
        axis=-1).reshape(b, rows, flat)
    w1 = jnp.concatenate(
        [jnp.zeros((1,), jnp.float32), time_w.reshape(-1),
         jnp.ones((d_model - 1 - nt,), jnp.float32)])
    b1 = jnp.concatenate(
        [jnp.zeros((1,), jnp.float32), time_b.reshape(-1),
         jnp.zeros((d_model - 1 - nt,), jnp.float32)])
    w640 = jnp.tile(w1, lpr)[None, :]
    b640 = jnp.tile(b1, lpr)[None, :]
    le_flat = local_emb[:length, :].reshape(rows, flat)
    vr = (input + space_emb.reshape(1, 1, d_input)).transpose(0, 2, 1) \
        .reshape(b, d_input, rows, lpr)

    out, vid = pl.pallas_call(
        _body,
        grid=(b,),
        in_specs=[
            pl.BlockSpec((1, d_input, rows, lpr), lambda bb: (bb, 0, 0, 0)),
            pl.BlockSpec((1, rows, flat), lambda bb: (bb, 0, 0)),
            pl.BlockSpec((1, flat), lambda bb: (0, 0)),
            pl.BlockSpec((1, flat), lambda bb: (0, 0)),
            pl.BlockSpec((rows, flat), lambda bb: (0, 0)),
        ],
        out_specs=[
            pl.BlockSpec((1, d_input * rows, flat), lambda bb: (bb, 0, 0)),
            pl.BlockSpec((b, d_input * length), lambda bb: (0, 0)),
        ],
        out_shape=[
            jax.ShapeDtypeStruct((b, d_input * rows, flat), jnp.float32),
            jax.ShapeDtypeStruct((b, d_input * length), jnp.int32),
        ],
        compiler_params=pltpu.CompilerParams(
            dimension_semantics=("arbitrary",)),
    )(vr, feats, w640, b640, le_flat)
    return out.reshape(b, d_input * length, d_model), vid


# speedup vs baseline: 1.3957x; 1.3957x over previous
import jax
import jax.numpy as jnp
import numpy as np
from jax.experimental import pallas as pl
from jax.experimental.pallas import tpu as pltpu

N_TIME, PER_DIM = 6, 6
LR = 128
NBUF = 4

_INV_PI = np.float32(0.3183098861837907)
_PI_HI = np.float32(3.140625)
_PI_LO = np.float32(9.676535897932795e-4)
_S3 = np.float32(-1.6665861e-01)
_S5 = np.float32(8.3121910e-03)
_S7 = np.float32(-1.8497128e-04)


def _fast_sin(x):
    kf = jnp.round(x * _INV_PI)
    r = x - kf * _PI_HI
    r = r - kf * _PI_LO
    r2 = r * r
    p = r + r * (r2 * (_S3 + r2 * (_S5 + r2 * _S7)))
    odd = jax.lax.shift_left(kf.astype(jnp.int32), np.int32(31))
    return jax.lax.bitcast_convert_type(
        jax.lax.bitcast_convert_type(p, jnp.int32) ^ odd, jnp.float32)


def _body(inp_ref, feat_ref, w_ref, b_ref, sp_ref, le_ref, out_ref, vid_ref,
          base_s, valsp_s, obuf, vbuf, sems, vsem):
    bb = pl.program_id(0)
    nb = pl.num_programs(0)
    lng = inp_ref.shape[1]
    c_dim = feat_ref.shape[2]
    d_in = inp_ref.shape[2]
    w_row = w_ref[...]
    b_row = b_ref[...]
    sp_row = sp_ref[...]
    c = jax.lax.broadcasted_iota(jnp.int32, (LR, c_dim), 1)
    sinsel = (c >= 1) & (c <= N_TIME * PER_DIM) & ((c - 1) % PER_DIM != 0)
    for lr in range(lng // LR):
        r0 = lr * LR
        xa = feat_ref[0, r0:r0 + LR, :] * w_row + b_row
        base_s[r0:r0 + LR, :] = le_ref[r0:r0 + LR, :] + jnp.where(
            sinsel, _fast_sin(xa), xa)
        valsp_s[r0:r0 + LR, :] = inp_ref[0, r0:r0 + LR, :] + sp_row

    def chunk_copy(k, dd):
        return pltpu.make_async_copy(
            obuf.at[k], out_ref.at[bb, pl.ds(dd * lng, lng), :], sems.at[k])

    for dd in range(d_in):
        k = dd % NBUF
        if dd >= NBUF:
            chunk_copy(k, dd - NBUF).wait()
        else:
            @pl.when(bb > 0)
            def _w():
                chunk_copy(k, d_in - NBUF + dd).wait()
        col = jax.lax.slice(valsp_s[...], (0, dd), (lng, dd + 1))
        obuf[k] = base_s[...] + jax.lax.pad(
            col, 0.0, ((0, 0, 0), (0, c_dim - 1, 0)))
        chunk_copy(k, dd).start()

    @pl.when(bb == 0)
    def _fill_vid():
        t = jax.lax.broadcasted_iota(jnp.int32, vbuf.shape, 1)
        vbuf[...] = t // lng
        pltpu.make_async_copy(vbuf, vid_ref, vsem).start()

    @pl.when(bb == nb - 1)
    def _drain():
        for dd in range(d_in - NBUF, d_in):
            chunk_copy(dd % NBUF, dd).wait()
        pltpu.make_async_copy(vbuf, vid_ref, vsem).wait()


def kernel(input, dates, cmax, time_w, time_b, local_emb, space_emb):
    b, length, d_input = input.shape
    d_model = local_emb.shape[1]
    n_time, per_dim = time_w.shape
    nt = n_time * per_dim
    feats = jnp.concatenate(
        [jnp.zeros((b, length, 1), jnp.float32),
         jnp.repeat(dates, per_dim, axis=-1), cmax], axis=-1)
    w40 = jnp.concatenate(
        [jnp.zeros((1,), jnp.float32), time_w.reshape(-1),
         jnp.ones((d_model - 1 - nt,), jnp.float32)])[None, :]
    b40 = jnp.concatenate(
        [jnp.zeros((1,), jnp.float32), time_b.reshape(-1),
         jnp.zeros((d_model - 1 - nt,), jnp.float32)])[None, :]

    return pl.pallas_call(
        _body,
        grid=(b,),
        in_specs=[
            pl.BlockSpec((1, length, d_input), lambda bb: (bb, 0, 0)),
            pl.BlockSpec((1, length, d_model), lambda bb: (bb, 0, 0)),
            pl.BlockSpec((1, d_model), lambda bb: (0, 0)),
            pl.BlockSpec((1, d_model), lambda bb: (0, 0)),
            pl.BlockSpec((1, d_input), lambda bb: (0, 0)),
            pl.BlockSpec((length, d_model), lambda bb: (0, 0)),
        ],
        out_specs=[
            pl.BlockSpec(memory_space=pltpu.MemorySpace.HBM),
            pl.BlockSpec(memory_space=pltpu.MemorySpace.HBM),
        ],
        out_shape=[
            jax.ShapeDtypeStruct((b, d_input * length, d_model), jnp.float32),
            jax.ShapeDtypeStruct((b, d_input * length), jnp.int32),
        ],
        scratch_shapes=[
            pltpu.VMEM((length, d_model), jnp.float32),
            pltpu.VMEM((length, d_input), jnp.float32),
            pltpu.VMEM((NBUF, length, d_model), jnp.float32),
            pltpu.VMEM((b, d_input * length), jnp.int32),
            pltpu.SemaphoreType.DMA((NBUF,)),
            pltpu.SemaphoreType.DMA,
        ],
        compiler_params=pltpu.CompilerParams(
            dimension_semantics=("arbitrary",)),
    )(input, feats, w40, b40, space_emb.reshape(1, d_input), local_emb)


# speedup vs baseline: 1.3972x; 1.0011x over previous
import jax
import jax.numpy as jnp
import numpy as np
from jax.experimental import pallas as pl
from jax.experimental.pallas import tpu as pltpu

N_TIME, PER_DIM = 6, 6
LR = 128

_MAGIC = np.float32(12582912.0)
_INV_PI = np.float32(0.3183098861837907)
_PI_HI = np.float32(3.140625)
_PI_LO = np.float32(9.676535897932795e-4)
_S3 = np.float32(-1.6665861e-01)
_S5 = np.float32(8.3121910e-03)
_S7 = np.float32(-1.8497128e-04)


def _fast_sin(x):
    kf = jnp.round(x * _INV_PI)
    r = x - kf * _PI_HI
    r = r - kf * _PI_LO
    r2 = r * r
    p = r + r * (r2 * (_S3 + r2 * (_S5 + r2 * _S7)))
    odd = jax.lax.shift_left(kf.astype(jnp.int32), np.int32(31))
    return jax.lax.bitcast_convert_type(
        jax.lax.bitcast_convert_type(p, jnp.int32) ^ odd, jnp.float32)


def _body(inp_ref, feat_ref, w_ref, b_ref, sp_ref, le_ref, out_ref, vid_ref):
    bb = pl.program_id(0)
    lng = inp_ref.shape[1]
    c_dim = feat_ref.shape[2]
    d_in = inp_ref.shape[2]
    w_row = w_ref[...]
    b_row = b_ref[...]
    sp_row = sp_ref[...]
    c = jax.lax.broadcasted_iota(jnp.int32, (LR, c_dim), 1)
    sinsel = (c >= 1) & (c <= N_TIME * PER_DIM) & ((c - 1) % PER_DIM != 0)
    for lr in range(lng // LR):
        r0 = lr * LR
        xa = feat_ref[0, r0:r0 + LR, :] * w_row + b_row
        basec = le_ref[r0:r0 + LR, :] + jnp.where(sinsel, _fast_sin(xa), xa)
        vspc = inp_ref[0, r0:r0 + LR, :] + sp_row
        for dd in range(d_in):
            col = jax.lax.slice(vspc, (0, dd), (LR, dd + 1))
            out_ref[0, dd * lng + r0:dd * lng + r0 + LR, :] = basec + \
                jax.lax.pad(col, 0.0, ((0, 0, 0), (0, c_dim - 1, 0)))

    @pl.when(bb == 0)
    def _fill_vid():
        t = jax.lax.broadcasted_iota(jnp.int32, vid_ref.shape, 1)
        vid_ref[...] = t // lng


def kernel(input, dates, cmax, time_w, time_b, local_emb, space_emb):
    b, length, d_input = input.shape
    d_model = local_emb.shape[1]
    n_time, per_dim = time_w.shape
    nt = n_time * per_dim
    feats = jnp.concatenate(
        [jnp.zeros((b, length, 1), jnp.float32),
         jnp.repeat(dates, per_dim, axis=-1), cmax], axis=-1)
    w40 = jnp.concatenate(
        [jnp.zeros((1,), jnp.float32), time_w.reshape(-1),
         jnp.ones((d_model - 1 - nt,), jnp.float32)])[None, :]
    b40 = jnp.concatenate(
        [jnp.zeros((1,), jnp.float32), time_b.reshape(-1),
         jnp.zeros((d_model - 1 - nt,), jnp.float32)])[None, :]

    return pl.pallas_call(
        _body,
        grid=(b,),
        in_specs=[
            pl.BlockSpec((1, length, d_input), lambda bb: (bb, 0, 0)),
            pl.BlockSpec((1, length, d_model), lambda bb: (bb, 0, 0)),
            pl.BlockSpec((1, d_model), lambda bb: (0, 0)),
            pl.BlockSpec((1, d_model), lambda bb: (0, 0)),
            pl.BlockSpec((1, d_input), lambda bb: (0, 0)),
            pl.BlockSpec((length, d_model), lambda bb: (0, 0)),
        ],
        out_specs=[
            pl.BlockSpec((1, d_input * length, d_model), lambda bb: (bb, 0, 0)),
            pl.BlockSpec((b, d_input * length), lambda bb: (0, 0)),
        ],
        out_shape=[
            jax.ShapeDtypeStruct((b, d_input * length, d_model), jnp.float32),
            jax.ShapeDtypeStruct((b, d_input * length), jnp.int32),
        ],
        compiler_params=pltpu.CompilerParams(
            dimension_semantics=("arbitrary",)),
    )(input, feats, w40, b40, space_emb.reshape(1, d_input), local_emb)
